# baseline (speedup 1.0000x reference)
import functools

import jax
import jax.numpy as jnp
from jax import lax
from jax.experimental import pallas as pl
from jax.experimental.pallas import tpu as pltpu
from jax.experimental.pallas import tpu_sc as plsc

BATCH = 4096
SEQ = 200
EMB = 64
LANES = 16
NUM_WORKERS = 32
B_PER_W = BATCH // NUM_WORKERS
C0 = 128
C1 = SEQ - C0


def _fire_gathers(table_hbm, idx_all, buf, sem, i):
    pltpu.async_copy(
        table_hbm.at[idx_all.at[i, pl.ds(0, C0)]], buf.at[pl.ds(0, C0)], sem
    )
    pltpu.async_copy(
        table_hbm.at[idx_all.at[i, pl.ds(C0, C1)]], buf.at[pl.ds(C0, C1)], sem
    )


def _wait_gathers(table_hbm, buf, sem):
    pltpu.make_async_copy(table_hbm.at[pl.ds(0, SEQ)], buf, sem).wait()


def _reduce_row(buf, out_v, dst):
    accs = tuple(buf[0, pl.ds(LANES * c, LANES)] for c in range(EMB // LANES))

    def body(r, accs):
        return tuple(
            jnp.maximum(accs[c], buf[r, pl.ds(LANES * c, LANES)])
            for c in range(EMB // LANES)
        )

    accs = lax.fori_loop(1, SEQ, body, accs, unroll=4)
    for c in range(EMB // LANES):
        out_v[dst, pl.ds(LANES * c, LANES)] = accs[c]


def _bow_encode(ids, table):
    mesh = plsc.VectorSubcoreMesh(core_axis_name="c", subcore_axis_name="s")

    @functools.partial(
        pl.kernel,
        out_type=jax.ShapeDtypeStruct((BATCH, EMB), jnp.float32),
        mesh=mesh,
        scratch_types=[
            pltpu.VMEM((B_PER_W, SEQ), jnp.int32),
            pltpu.VMEM((SEQ, EMB), jnp.float32),
            pltpu.VMEM((SEQ, EMB), jnp.float32),
            pltpu.VMEM((B_PER_W, EMB), jnp.float32),
            pltpu.SemaphoreType.DMA,
            pltpu.SemaphoreType.DMA,
        ],
        compiler_params=pltpu.CompilerParams(use_tc_tiling_on_sc=False),
    )
    def k(ids_hbm, table_hbm, out_hbm, idx_all, buf_a, buf_b, out_v, sem_a, sem_b):
        wid = lax.axis_index("s") * 2 + lax.axis_index("c")
        base = wid * B_PER_W

        pltpu.sync_copy(ids_hbm.at[pl.ds(base, B_PER_W)], idx_all)
        _fire_gathers(table_hbm, idx_all, buf_a, sem_a, 0)

        def body(j, _):
            i = j * 2
            _fire_gathers(table_hbm, idx_all, buf_b, sem_b, i + 1)
            _wait_gathers(table_hbm, buf_a, sem_a)
            _reduce_row(buf_a, out_v, i)

            @pl.when(i + 2 < B_PER_W)
            def _():
                _fire_gathers(table_hbm, idx_all, buf_a, sem_a, i + 2)

            _wait_gathers(table_hbm, buf_b, sem_b)
            _reduce_row(buf_b, out_v, i + 1)
            return 0

        lax.fori_loop(0, B_PER_W // 2, body, 0)
        pltpu.sync_copy(out_v, out_hbm.at[pl.ds(base, B_PER_W)])

    return k(ids, table)


def kernel(input, emb_weight):
    ids = jnp.asarray(input, jnp.int32)
    return _bow_encode(ids, emb_weight)


# speedup vs baseline: 1.0578x; 1.0578x over previous
import functools

import jax
import jax.numpy as jnp
from jax import lax
from jax.experimental import pallas as pl
from jax.experimental.pallas import tpu as pltpu
from jax.experimental.pallas import tpu_sc as plsc

BATCH = 4096
SEQ = 200
EMB = 64
LANES = 16
NUM_WORKERS = 32
B_PER_W = BATCH // NUM_WORKERS
C0 = 128
C1 = SEQ - C0


def _fire_gathers(table_hbm, idx_all, buf, sem, i):
    pltpu.async_copy(
        table_hbm.at[idx_all.at[i, pl.ds(0, C0)]], buf.at[pl.ds(0, C0)], sem
    )
    pltpu.async_copy(
        table_hbm.at[idx_all.at[i, pl.ds(C0, C1)]], buf.at[pl.ds(C0, C1)], sem
    )


def _wait_gathers(table_hbm, buf, sem):
    pltpu.make_async_copy(table_hbm.at[pl.ds(0, SEQ)], buf, sem).wait()


def _reduce_row(buf, out_v, dst):
    accs = tuple(buf[0, pl.ds(LANES * c, LANES)] for c in range(EMB // LANES))

    def body(r, accs):
        return tuple(
            jnp.maximum(accs[c], buf[r, pl.ds(LANES * c, LANES)])
            for c in range(EMB // LANES)
        )

    accs = lax.fori_loop(1, SEQ, body, accs, unroll=8)
    for c in range(EMB // LANES):
        out_v[dst, pl.ds(LANES * c, LANES)] = accs[c]


def _bow_encode(ids, table):
    mesh = plsc.VectorSubcoreMesh(core_axis_name="c", subcore_axis_name="s")

    @functools.partial(
        pl.kernel,
        out_type=jax.ShapeDtypeStruct((BATCH, EMB), jnp.float32),
        mesh=mesh,
        scratch_types=[
            pltpu.VMEM((B_PER_W, SEQ), jnp.int32),
            pltpu.VMEM((SEQ, EMB), jnp.float32),
            pltpu.VMEM((SEQ, EMB), jnp.float32),
            pltpu.VMEM((SEQ, EMB), jnp.float32),
            pltpu.VMEM((SEQ, EMB), jnp.float32),
            pltpu.VMEM((B_PER_W, EMB), jnp.float32),
            pltpu.SemaphoreType.DMA,
            pltpu.SemaphoreType.DMA,
            pltpu.SemaphoreType.DMA,
            pltpu.SemaphoreType.DMA,
        ],
        compiler_params=pltpu.CompilerParams(use_tc_tiling_on_sc=False),
    )
    def k(ids_hbm, table_hbm, out_hbm, idx_all, b0, b1, b2, b3, out_v,
          s0, s1, s2, s3):
        wid = lax.axis_index("s") * 2 + lax.axis_index("c")
        base = wid * B_PER_W
        bufs = (b0, b1, b2, b3)
        sems = (s0, s1, s2, s3)
        nbuf = len(bufs)

        pltpu.sync_copy(ids_hbm.at[pl.ds(base, B_PER_W)], idx_all)
        for r in range(nbuf - 1):
            _fire_gathers(table_hbm, idx_all, bufs[r], sems[r], r)

        def body(j, _):
            i = j * nbuf
            for b in range(nbuf):
                row = i + b
                nxt = row + nbuf - 1
                bn = (b + nbuf - 1) % nbuf

                @pl.when(nxt < B_PER_W)
                def _():
                    _fire_gathers(table_hbm, idx_all, bufs[bn], sems[bn], nxt)

                _wait_gathers(table_hbm, bufs[b], sems[b])
                _reduce_row(bufs[b], out_v, row)
            return 0

        lax.fori_loop(0, B_PER_W // nbuf, body, 0)
        pltpu.sync_copy(out_v, out_hbm.at[pl.ds(base, B_PER_W)])

    return k(ids, table)


def kernel(input, emb_weight):
    ids = jnp.asarray(input, jnp.int32)
    return _bow_encode(ids, emb_weight)


# speedup vs baseline: 1.0585x; 1.0006x over previous
import functools

import jax
import jax.numpy as jnp
from jax import lax
from jax.experimental import pallas as pl
from jax.experimental.pallas import tpu as pltpu
from jax.experimental.pallas import tpu_sc as plsc

BATCH = 4096
SEQ = 200
EMB = 64
LANES = 16
NUM_WORKERS = 32
B_PER_W = BATCH // NUM_WORKERS
ROWS_PER_CHUNK = 2
CHUNK = ROWS_PER_CHUNK * SEQ
N_CHUNKS = B_PER_W // ROWS_PER_CHUNK
NBUF = 3


def _fire_gather(table_hbm, idx_all, buf, sem, c):
    pltpu.async_copy(table_hbm.at[idx_all.at[c]], buf, sem)


def _wait_gather(table_hbm, buf, sem):
    pltpu.make_async_copy(table_hbm.at[pl.ds(0, CHUNK)], buf, sem).wait()


def _reduce_row(buf, r0, out_v, dst):
    accs = tuple(buf[r0, pl.ds(LANES * c, LANES)] for c in range(EMB // LANES))

    def body(r, accs):
        return tuple(
            jnp.maximum(accs[c], buf[r, pl.ds(LANES * c, LANES)])
            for c in range(EMB // LANES)
        )

    accs = lax.fori_loop(r0 + 1, r0 + SEQ, body, accs, unroll=8)
    for c in range(EMB // LANES):
        out_v[dst, pl.ds(LANES * c, LANES)] = accs[c]


def _bow_encode(ids2, table):
    mesh = plsc.VectorSubcoreMesh(core_axis_name="c", subcore_axis_name="s")

    @functools.partial(
        pl.kernel,
        out_type=jax.ShapeDtypeStruct((BATCH, EMB), jnp.float32),
        mesh=mesh,
        scratch_types=[
            pltpu.VMEM((N_CHUNKS, CHUNK), jnp.int32),
            pltpu.VMEM((CHUNK, EMB), jnp.float32),
            pltpu.VMEM((CHUNK, EMB), jnp.float32),
            pltpu.VMEM((CHUNK, EMB), jnp.float32),
            pltpu.VMEM((B_PER_W, EMB), jnp.float32),
            pltpu.SemaphoreType.DMA,
            pltpu.SemaphoreType.DMA,
            pltpu.SemaphoreType.DMA,
        ],
        compiler_params=pltpu.CompilerParams(use_tc_tiling_on_sc=False),
    )
    def k(ids_hbm, table_hbm, out_hbm, idx_all, b0, b1, b2, out_v, s0, s1, s2):
        wid = lax.axis_index("s") * 2 + lax.axis_index("c")
        base = wid * B_PER_W
        bufs = (b0, b1, b2)
        sems = (s0, s1, s2)

        pltpu.sync_copy(ids_hbm.at[pl.ds(N_CHUNKS * wid, N_CHUNKS)], idx_all)
        for c in range(NBUF - 1):
            _fire_gather(table_hbm, idx_all, bufs[c], sems[c], c)

        def process(chunk, b):
            _wait_gather(table_hbm, bufs[b], sems[b])
            for r in range(ROWS_PER_CHUNK):
                _reduce_row(bufs[b], r * SEQ, out_v, chunk * ROWS_PER_CHUNK + r)

        def body(j, _):
            c = j * NBUF
            for b in range(NBUF):
                bn = (b + NBUF - 1) % NBUF

                @pl.when(c + b + NBUF - 1 < N_CHUNKS)
                def _():
                    _fire_gather(
                        table_hbm, idx_all, bufs[bn], sems[bn], c + b + NBUF - 1
                    )

                process(c + b, b)
            return 0

        lax.fori_loop(0, N_CHUNKS // NBUF, body, 0)
        for c in range(N_CHUNKS - N_CHUNKS % NBUF, N_CHUNKS):
            process(c, c % NBUF)
        pltpu.sync_copy(out_v, out_hbm.at[pl.ds(base, B_PER_W)])

    return k(ids2, table)


def kernel(input, emb_weight):
    ids2 = jnp.asarray(input, jnp.int32).reshape(BATCH // ROWS_PER_CHUNK, CHUNK)
    return _bow_encode(ids2, emb_weight)
